# baseline (speedup 1.0000x reference)
import functools

import jax
import jax.numpy as jnp
from jax.experimental import pallas as pl
from jax.experimental.pallas import tpu as pltpu

_B = 8
_PE_G = 6
_PE_K = 8
_ENC_CH = (96, 192, 384, 512)
_ENC_G = (12, 24, 48, 64)
_ENC_K = (16, 16, 16, 16)
_GRIDS = (0.06, 0.15, 0.375, 0.9375)
_SZ = (1024, 384, 32, 8)


def _linear_kern(x_ref, w_ref, b_ref, o_ref, *, relu):
    y = jnp.dot(x_ref[...], w_ref[...], preferred_element_type=jnp.float32)
    y = y + b_ref[...]
    if relu:
        y = jnp.maximum(y, 0.0)
    o_ref[...] = y


def _plinear(x, w, b, relu=False, tile=512):
    n, din = x.shape
    dout = w.shape[1]
    tile = min(tile, n)
    while n % tile:
        tile //= 2
    return pl.pallas_call(
        functools.partial(_linear_kern, relu=relu),
        grid=(n // tile,),
        in_specs=[
            pl.BlockSpec((tile, din), lambda i: (i, 0)),
            pl.BlockSpec((din, dout), lambda i: (0, 0)),
            pl.BlockSpec((1, dout), lambda i: (0, 0)),
        ],
        out_specs=pl.BlockSpec((tile, dout), lambda i: (i, 0)),
        out_shape=jax.ShapeDtypeStruct((n, dout), jnp.float32),
    )(x, w, b.reshape(1, -1))


def _knn_kern(cnt_ref, c_ref, caT_ref, o_ref, *, K, SZ):
    b = pl.program_id(0)
    cnt = cnt_ref[b]
    rows = c_ref[...]
    RT = rows.shape[0]
    cx = caT_ref[0, 0:1, :]
    cy = caT_ref[0, 1:2, :]
    cz = caT_ref[0, 2:3, :]
    dx = rows[:, 0:1] - cx
    dy = rows[:, 1:2] - cy
    dz = rows[:, 2:3] - cz
    d = (dx * dx + dy * dy) + dz * dz
    jl = jax.lax.broadcasted_iota(jnp.int32, (RT, SZ), 1)
    d = jnp.where(jl < cnt, d, jnp.inf)
    cols = []
    for _ in range(K):
        m = jnp.min(d, axis=1, keepdims=True)
        ismin = d == m
        fidx = jnp.min(jnp.where(ismin, jl, SZ), axis=1, keepdims=True)
        cols.append(fidx)
        d = jnp.where(jl == fidx, jnp.inf, d)
    out = jnp.concatenate(cols, axis=1).astype(jnp.int32) + b * SZ
    o_ref[...] = out


def _knn_pallas(c, cnts, bidv, validv, bcount, k, sz):
    m = c.shape[0]
    rt = min(128, sz)
    tt = sz // rt
    caT = c.reshape(_B, sz, 3).transpose(0, 2, 1)
    nb = pl.pallas_call(
        functools.partial(_knn_kern, K=k, SZ=sz),
        grid=(_B, tt),
        in_specs=[
            pl.BlockSpec(memory_space=pltpu.MemorySpace.SMEM),
            pl.BlockSpec((rt, 3), lambda b, t: (b * tt + t, 0)),
            pl.BlockSpec((1, 3, sz), lambda b, t: (b, 0, 0)),
        ],
        out_specs=pl.BlockSpec((rt, k), lambda b, t: (b * tt + t, 0)),
        out_shape=jax.ShapeDtypeStruct((m, k), jnp.int32),
    )(cnts, c, caT)
    vcnt = bcount[jnp.where(validv, bidv, 0)]
    col = jnp.arange(k)[None, :]
    last = jnp.take_along_axis(nb, jnp.clip(vcnt - 1, 0, k - 1)[:, None], axis=1)
    return jnp.where(col < vcnt[:, None], nb, last)


def _gva_kern(q_ref, feat_ref, co_ref, nbr_ref, ktab_ref, vtab_ref, cw_ref,
              pe1_ref, pe1b_ref, pe2_ref, pe2b_ref,
              we1_ref, we1b_ref, we2_ref, we2b_ref,
              gmat_ref, proj_ref, projb_ref, o_ref, *, K, G, SZ):
    TN, ch = q_ref.shape
    TNK = TN * K
    nl = nbr_ref[...]
    oh = (jax.lax.broadcasted_iota(jnp.int32, (TNK, SZ), 1) == nl)
    oh = oh.astype(jnp.float32)
    kn = jnp.dot(oh, ktab_ref[...], preferred_element_type=jnp.float32)
    vn = jnp.dot(oh, vtab_ref[...], preferred_element_type=jnp.float32)
    cn = jnp.dot(oh, cw_ref[...], preferred_element_type=jnp.float32)
    pos = (cn.reshape(TN, K, 3) - co_ref[...].reshape(TN, 1, 3)).reshape(TNK, 3)
    h = jnp.dot(pos, pe1_ref[...], preferred_element_type=jnp.float32)
    h = jnp.maximum(h + pe1b_ref[...], 0.0)
    pe = jnp.dot(h, pe2_ref[...], preferred_element_type=jnp.float32)
    pe = pe + pe2b_ref[...]
    rel = (q_ref[...].reshape(TN, 1, ch)
           - kn.reshape(TN, K, ch)
           + pe.reshape(TN, K, ch)).reshape(TNK, ch)
    t = jnp.dot(rel, we1_ref[...], preferred_element_type=jnp.float32)
    t = jnp.maximum(t + we1b_ref[...], 0.0)
    w = jnp.dot(t, we2_ref[...], preferred_element_type=jnp.float32)
    w = (w + we2b_ref[...]).reshape(TN, K, G)
    w = w - jnp.max(w, axis=1, keepdims=True)
    e = jnp.exp(w)
    w = e / jnp.sum(e, axis=1, keepdims=True)
    wfull = jnp.dot(w.reshape(TNK, G), gmat_ref[...],
                    preferred_element_type=jnp.float32)
    val = vn + pe
    out = jnp.sum((wfull * val).reshape(TN, K, ch), axis=1)
    y = jnp.dot(out, proj_ref[...], preferred_element_type=jnp.float32)
    y = y + projb_ref[...]
    o_ref[...] = feat_ref[...] + jnp.maximum(y, 0.0)


def _gva(p, feat, coord, nbr, g, sz):
    n, ch = feat.shape
    k = nbr.shape[1]
    wqkv = jnp.concatenate([p["q"]["w"], p["k"]["w"], p["v"]["w"]], axis=1)
    bqkv = jnp.concatenate([p["q"]["b"], p["k"]["b"], p["v"]["b"]])
    qkv = _plinear(feat, wqkv, bqkv)
    q = qkv[:, :ch]
    ktab = qkv[:, ch:2 * ch]
    vtab = qkv[:, 2 * ch:]
    nloc = (nbr - (jnp.arange(n, dtype=jnp.int32)[:, None] // sz) * sz
            ).reshape(n * k, 1)
    gmat = (jnp.arange(ch)[None, :] // (ch // g)
            == jnp.arange(g)[:, None]).astype(jnp.float32)
    TN = min(128, sz)
    tpb = sz // TN
    row = lambda r, c: pl.BlockSpec((r, c), lambda i: (i, 0))
    win = lambda r, c: pl.BlockSpec((r, c), lambda i: (i // tpb, 0))
    full = lambda r, c: pl.BlockSpec((r, c), lambda i: (0, 0))
    return pl.pallas_call(
        functools.partial(_gva_kern, K=k, G=g, SZ=sz),
        grid=(n // TN,),
        in_specs=[
            row(TN, ch), row(TN, ch), row(TN, 3), row(TN * k, 1),
            win(sz, ch), win(sz, ch), win(sz, 3),
            full(3, ch), full(1, ch), full(ch, ch), full(1, ch),
            full(ch, ch), full(1, ch), full(ch, g), full(1, g),
            full(g, ch), full(ch, ch), full(1, ch),
        ],
        out_specs=row(TN, ch),
        out_shape=jax.ShapeDtypeStruct((n, ch), jnp.float32),
    )(q, feat, coord, nloc, ktab, vtab, coord,
      p["pe1"]["w"], p["pe1"]["b"].reshape(1, -1),
      p["pe2"]["w"], p["pe2"]["b"].reshape(1, -1),
      p["we1"]["w"], p["we1"]["b"].reshape(1, -1),
      p["we2"]["w"], p["we2"]["b"].reshape(1, -1),
      gmat, p["proj"]["w"], p["proj"]["b"].reshape(1, -1))


def _grid_cluster(c, bidv, validv, grid):
    n = c.shape[0]
    bid_safe = jnp.where(validv, bidv, 0)
    bmin = jax.ops.segment_min(jnp.where(validv[:, None], c, jnp.inf),
                               bid_safe, num_segments=_B)
    v = jnp.floor((c - bmin[bid_safe]) / grid).astype(jnp.int32)
    keyv = v[:, 0] * 1000000 + v[:, 1] * 1000 + v[:, 2]
    bkey = jnp.where(validv, bidv, _B)
    order = jnp.lexsort((keyv, bkey))
    sk = keyv[order]
    sb = bkey[order]
    flag = jnp.concatenate([
        jnp.zeros(1, jnp.int32),
        ((sk[1:] != sk[:-1]) | (sb[1:] != sb[:-1])).astype(jnp.int32)])
    ids = jnp.cumsum(flag)
    return jnp.zeros(n, jnp.int32).at[order].set(ids)


def _cluskey_kern(cnt_ref, c_ref, caT_ref, key_ref, rep_ref, *, SZ, RT, GRID):
    b = pl.program_id(0)
    t = pl.program_id(1)
    cnt = cnt_ref[b]
    rows = c_ref[...]
    cx = caT_ref[0, 0:1, :]
    cy = caT_ref[0, 1:2, :]
    cz = caT_ref[0, 2:3, :]
    lane = jax.lax.broadcasted_iota(jnp.int32, (1, SZ), 1)
    lv = lane < cnt
    bmx = jnp.min(jnp.where(lv, cx, jnp.inf), axis=1, keepdims=True)
    bmy = jnp.min(jnp.where(lv, cy, jnp.inf), axis=1, keepdims=True)
    bmz = jnp.min(jnp.where(lv, cz, jnp.inf), axis=1, keepdims=True)
    kw = (jnp.floor((cx - bmx) / GRID).astype(jnp.int32) * 1000000
          + jnp.floor((cy - bmy) / GRID).astype(jnp.int32) * 1000
          + jnp.floor((cz - bmz) / GRID).astype(jnp.int32))
    kr = (jnp.floor((rows[:, 0:1] - bmx) / GRID).astype(jnp.int32) * 1000000
          + jnp.floor((rows[:, 1:2] - bmy) / GRID).astype(jnp.int32) * 1000
          + jnp.floor((rows[:, 2:3] - bmz) / GRID).astype(jnp.int32))
    rloc = t * RT + jax.lax.broadcasted_iota(jnp.int32, (RT, 1), 0)
    dup = lv & (lane < rloc) & (kw == kr)
    rep = jnp.logical_not(jnp.any(dup, axis=1, keepdims=True))
    rep = rep & (rloc < cnt)
    key_ref[...] = kr
    rep_ref[...] = rep.astype(jnp.int32)


def _clusrank_kern(cnt_ref, key_ref, keyT_ref, repT_ref, rank_ref, *, SZ):
    b = pl.program_id(0)
    cnt = cnt_ref[b]
    kr = key_ref[...]
    kw = keyT_ref[0, 0:1, :]
    rw = repT_ref[0, 0:1, :]
    lane = jax.lax.broadcasted_iota(jnp.int32, (1, SZ), 1)
    cmp = (lane < cnt) & (rw > 0) & (kw < kr)
    rank_ref[...] = jnp.sum(cmp.astype(jnp.int32), axis=1, keepdims=True)


def _cluster_aligned(cur_c, cnt_in, sz_in, sz_out, grid):
    mm = cur_c.shape[0]
    rt = min(128, sz_in)
    tt = sz_in // rt
    caT = cur_c.reshape(_B, sz_in, 3).transpose(0, 2, 1)
    key, rep = pl.pallas_call(
        functools.partial(_cluskey_kern, SZ=sz_in, RT=rt, GRID=grid),
        grid=(_B, tt),
        in_specs=[
            pl.BlockSpec(memory_space=pltpu.MemorySpace.SMEM),
            pl.BlockSpec((rt, 3), lambda b, t: (b * tt + t, 0)),
            pl.BlockSpec((1, 3, sz_in), lambda b, t: (b, 0, 0)),
        ],
        out_specs=[pl.BlockSpec((rt, 1), lambda b, t: (b * tt + t, 0)),
                   pl.BlockSpec((rt, 1), lambda b, t: (b * tt + t, 0))],
        out_shape=[jax.ShapeDtypeStruct((mm, 1), jnp.int32),
                   jax.ShapeDtypeStruct((mm, 1), jnp.int32)],
    )(cnt_in, cur_c, caT)
    keyT = key.reshape(_B, 1, sz_in)
    repT = rep.reshape(_B, 1, sz_in)
    rank = pl.pallas_call(
        functools.partial(_clusrank_kern, SZ=sz_in),
        grid=(_B, tt),
        in_specs=[
            pl.BlockSpec(memory_space=pltpu.MemorySpace.SMEM),
            pl.BlockSpec((rt, 1), lambda b, t: (b * tt + t, 0)),
            pl.BlockSpec((1, 1, sz_in), lambda b, t: (b, 0, 0)),
            pl.BlockSpec((1, 1, sz_in), lambda b, t: (b, 0, 0)),
        ],
        out_specs=pl.BlockSpec((rt, 1), lambda b, t: (b * tt + t, 0)),
        out_shape=jax.ShapeDtypeStruct((mm, 1), jnp.int32),
    )(cnt_in, key, keyT, repT)
    idx = jnp.arange(mm, dtype=jnp.int32)
    bid = idx // sz_in
    validv = (idx % sz_in) < cnt_in[bid]
    clus = jnp.where(validv, bid * sz_out + rank[:, 0], _B * sz_out)
    seg_bcnt = jnp.sum(rep.reshape(_B, sz_in), axis=1, dtype=jnp.int32)
    rk = rank[:, 0].reshape(_B, sz_in)
    lv = jnp.arange(sz_in)[None, :] < cnt_in[:, None]
    oh = ((rk[:, None, :] == jnp.arange(sz_out)[None, :, None])
          & lv[:, None, :]).astype(jnp.float32)
    sums = jnp.einsum('bsq,bqc->bsc', oh, cur_c.reshape(_B, sz_in, 3),
                      precision=jax.lax.Precision.HIGHEST,
                      preferred_element_type=jnp.float32)
    szs = jnp.sum(oh, axis=2)
    seg_valid = jnp.arange(sz_out)[None, :] < seg_bcnt[:, None]
    cnt = jnp.where(seg_valid, szs, 1.0)
    pooled = (sums / cnt[..., None]).reshape(_B * sz_out, 3)
    return (clus, seg_bcnt, pooled, cnt.reshape(-1),
            seg_valid.reshape(-1))


def _structures(coord, offset):
    n = coord.shape[0]
    bid0 = jnp.searchsorted(offset, jnp.arange(n), side="right").astype(jnp.int32)
    valid0 = jnp.ones(n, dtype=bool)
    starts0 = jnp.concatenate([jnp.zeros(1, jnp.int32), offset[:-1]])
    bcnt0 = offset - starts0
    nbr0 = _knn_pallas(coord, bcnt0, bid0, valid0, bcnt0, _PE_K, _SZ[0])
    stages = []
    cur_c, cur_v, cnt_in, sz_in = coord, valid0, bcnt0, _SZ[0]
    for i in range(4):
        sz = _SZ[i]
        m = _B * sz
        clus, seg_bcnt, pooled_c, cnt, seg_valid = _cluster_aligned(
            cur_c, cnt_in, sz_in, sz, _GRIDS[i])
        seg_bid = jnp.where(seg_valid,
                            jnp.arange(m, dtype=jnp.int32) // sz, 0)
        nbr = _knn_pallas(pooled_c, seg_bcnt, seg_bid, seg_valid, seg_bcnt,
                          _ENC_K[i], sz)
        stages.append((clus, m, cnt, nbr, seg_valid, pooled_c))
        cur_c, cur_v, cnt_in, sz_in = pooled_c, seg_valid, seg_bcnt, sz
    bid = jnp.where(cur_v, jnp.arange(_B * _SZ[3], dtype=jnp.int32) // _SZ[3], 0)
    bcnt = cnt_in.astype(jnp.float32)
    return nbr0, stages, bid, bcnt


def kernel(coord, feat, offset, params):
    nbr0, stages, bid, bcnt = _structures(coord, offset)
    x = coord
    f = _plinear(feat, params["embed"]["w"], params["embed"]["b"])
    f = _gva(params["pe_block"], f, x, nbr0, _PE_G, _SZ[0])
    for i in range(4):
        f = _plinear(f, params["down"][i]["w"], params["down"][i]["b"],
                     relu=True)
        clus, nseg, cnt, nbr, seg_valid, px = stages[i]
        pf = jax.ops.segment_max(f, clus, num_segments=nseg)
        pf = jnp.where(seg_valid[:, None], pf, 0.0)
        f = _gva(params["blocks"][i], pf, px, nbr, _ENC_G[i], _SZ[i])
        x = px
    f = jnp.where(stages[-1][4][:, None], f, 0.0)
    pooled = jax.ops.segment_sum(f, bid, num_segments=bcnt.shape[0]) / bcnt[:, None]
    h = _plinear(pooled, params["head1"]["w"], params["head1"]["b"], relu=True)
    h = _plinear(h, params["head2"]["w"], params["head2"]["b"], relu=True)
    return _plinear(h, params["head3"]["w"], params["head3"]["b"])


# speedup vs baseline: 1.0734x; 1.0734x over previous
import functools

import jax
import jax.numpy as jnp
from jax.experimental import pallas as pl
from jax.experimental.pallas import tpu as pltpu

_B = 8
_PE_G = 6
_PE_K = 8
_ENC_CH = (96, 192, 384, 512)
_ENC_G = (12, 24, 48, 64)
_ENC_K = (16, 16, 16, 16)
_GRIDS = (0.06, 0.15, 0.375, 0.9375)
_SZ = (1024, 384, 32, 8)


def _linear_kern(x_ref, w_ref, b_ref, o_ref, *, relu):
    y = jnp.dot(x_ref[...], w_ref[...], preferred_element_type=jnp.float32)
    y = y + b_ref[...]
    if relu:
        y = jnp.maximum(y, 0.0)
    o_ref[...] = y


def _plinear(x, w, b, relu=False, tile=512):
    n, din = x.shape
    dout = w.shape[1]
    tile = min(tile, n)
    while n % tile:
        tile //= 2
    return pl.pallas_call(
        functools.partial(_linear_kern, relu=relu),
        grid=(n // tile,),
        in_specs=[
            pl.BlockSpec((tile, din), lambda i: (i, 0)),
            pl.BlockSpec((din, dout), lambda i: (0, 0)),
            pl.BlockSpec((1, dout), lambda i: (0, 0)),
        ],
        out_specs=pl.BlockSpec((tile, dout), lambda i: (i, 0)),
        out_shape=jax.ShapeDtypeStruct((n, dout), jnp.float32),
    )(x, w, b.reshape(1, -1))


def _head_kern(x_ref, w1_ref, b1_ref, w2_ref, b2_ref, w3_ref, b3_ref, o_ref):
    h = jnp.dot(x_ref[...], w1_ref[...], preferred_element_type=jnp.float32)
    h = jnp.maximum(h + b1_ref[...], 0.0)
    h = jnp.dot(h, w2_ref[...], preferred_element_type=jnp.float32)
    h = jnp.maximum(h + b2_ref[...], 0.0)
    h = jnp.dot(h, w3_ref[...], preferred_element_type=jnp.float32)
    o_ref[...] = h + b3_ref[...]


def _phead(x, p1, p2, p3):
    n = x.shape[0]
    dout = p3["w"].shape[1]
    return pl.pallas_call(
        _head_kern,
        out_shape=jax.ShapeDtypeStruct((n, dout), jnp.float32),
    )(x, p1["w"], p1["b"].reshape(1, -1), p2["w"], p2["b"].reshape(1, -1),
      p3["w"], p3["b"].reshape(1, -1))


def _knn_kern(cnt_ref, c_ref, caT_ref, o_ref, *, K, SZ):
    b = pl.program_id(0)
    cnt = cnt_ref[b]
    rows = c_ref[...]
    RT = rows.shape[0]
    cx = caT_ref[0, 0:1, :]
    cy = caT_ref[0, 1:2, :]
    cz = caT_ref[0, 2:3, :]
    dx = rows[:, 0:1] - cx
    dy = rows[:, 1:2] - cy
    dz = rows[:, 2:3] - cz
    d = (dx * dx + dy * dy) + dz * dz
    jl = jax.lax.broadcasted_iota(jnp.int32, (RT, SZ), 1)
    d = jnp.where(jl < cnt, d, jnp.inf)
    cols = []
    for _ in range(K):
        m = jnp.min(d, axis=1, keepdims=True)
        ismin = d == m
        fidx = jnp.min(jnp.where(ismin, jl, SZ), axis=1, keepdims=True)
        cols.append(fidx)
        d = jnp.where(jl == fidx, jnp.inf, d)
    out = jnp.concatenate(cols, axis=1).astype(jnp.int32) + b * SZ
    o_ref[...] = out


def _knn_pallas(c, cnts, bidv, validv, bcount, k, sz):
    m = c.shape[0]
    rt = min(256, sz)
    while sz % rt:
        rt //= 2
    tt = sz // rt
    caT = c.reshape(_B, sz, 3).transpose(0, 2, 1)
    nb = pl.pallas_call(
        functools.partial(_knn_kern, K=k, SZ=sz),
        grid=(_B, tt),
        in_specs=[
            pl.BlockSpec(memory_space=pltpu.MemorySpace.SMEM),
            pl.BlockSpec((rt, 3), lambda b, t: (b * tt + t, 0)),
            pl.BlockSpec((1, 3, sz), lambda b, t: (b, 0, 0)),
        ],
        out_specs=pl.BlockSpec((rt, k), lambda b, t: (b * tt + t, 0)),
        out_shape=jax.ShapeDtypeStruct((m, k), jnp.int32),
    )(cnts, c, caT)
    vcnt = bcount[jnp.where(validv, bidv, 0)]
    col = jnp.arange(k)[None, :]
    last = jnp.take_along_axis(nb, jnp.clip(vcnt - 1, 0, k - 1)[:, None], axis=1)
    return jnp.where(col < vcnt[:, None], nb, last)


def _gva_kern(q_ref, feat_ref, co_ref, nbr_ref, ktab_ref, vtab_ref, cw_ref,
              pe1_ref, pe1b_ref, pe2_ref, pe2b_ref,
              we1_ref, we1b_ref, we2_ref, we2b_ref,
              gmat_ref, proj_ref, projb_ref, o_ref, *, K, G, SZ):
    TN, ch = q_ref.shape
    TNK = TN * K
    nl = nbr_ref[...]
    oh = (jax.lax.broadcasted_iota(jnp.int32, (TNK, SZ), 1) == nl)
    oh = oh.astype(jnp.float32)
    kn = jnp.dot(oh, ktab_ref[...], preferred_element_type=jnp.float32)
    vn = jnp.dot(oh, vtab_ref[...], preferred_element_type=jnp.float32)
    cn = jnp.dot(oh, cw_ref[...], preferred_element_type=jnp.float32)
    pos = (cn.reshape(TN, K, 3) - co_ref[...].reshape(TN, 1, 3)).reshape(TNK, 3)
    h = jnp.dot(pos, pe1_ref[...], preferred_element_type=jnp.float32)
    h = jnp.maximum(h + pe1b_ref[...], 0.0)
    pe = jnp.dot(h, pe2_ref[...], preferred_element_type=jnp.float32)
    pe = pe + pe2b_ref[...]
    rel = (q_ref[...].reshape(TN, 1, ch)
           - kn.reshape(TN, K, ch)
           + pe.reshape(TN, K, ch)).reshape(TNK, ch)
    t = jnp.dot(rel, we1_ref[...], preferred_element_type=jnp.float32)
    t = jnp.maximum(t + we1b_ref[...], 0.0)
    w = jnp.dot(t, we2_ref[...], preferred_element_type=jnp.float32)
    w = (w + we2b_ref[...]).reshape(TN, K, G)
    w = w - jnp.max(w, axis=1, keepdims=True)
    e = jnp.exp(w)
    w = e / jnp.sum(e, axis=1, keepdims=True)
    wfull = jnp.dot(w.reshape(TNK, G), gmat_ref[...],
                    preferred_element_type=jnp.float32)
    val = vn + pe
    out = jnp.sum((wfull * val).reshape(TN, K, ch), axis=1)
    y = jnp.dot(out, proj_ref[...], preferred_element_type=jnp.float32)
    y = y + projb_ref[...]
    o_ref[...] = feat_ref[...] + jnp.maximum(y, 0.0)


def _gva(p, feat, coord, nbr, g, sz):
    n, ch = feat.shape
    k = nbr.shape[1]
    wqkv = jnp.concatenate([p["q"]["w"], p["k"]["w"], p["v"]["w"]], axis=1)
    bqkv = jnp.concatenate([p["q"]["b"], p["k"]["b"], p["v"]["b"]])
    qkv = _plinear(feat, wqkv, bqkv)
    q = qkv[:, :ch]
    ktab = qkv[:, ch:2 * ch]
    vtab = qkv[:, 2 * ch:]
    nloc = (nbr - (jnp.arange(n, dtype=jnp.int32)[:, None] // sz) * sz
            ).reshape(n * k, 1)
    gmat = (jnp.arange(ch)[None, :] // (ch // g)
            == jnp.arange(g)[:, None]).astype(jnp.float32)
    TN = min(128, sz)
    tpb = sz // TN
    row = lambda r, c: pl.BlockSpec((r, c), lambda i: (i, 0))
    win = lambda r, c: pl.BlockSpec((r, c), lambda i: (i // tpb, 0))
    full = lambda r, c: pl.BlockSpec((r, c), lambda i: (0, 0))
    return pl.pallas_call(
        functools.partial(_gva_kern, K=k, G=g, SZ=sz),
        grid=(n // TN,),
        in_specs=[
            row(TN, ch), row(TN, ch), row(TN, 3), row(TN * k, 1),
            win(sz, ch), win(sz, ch), win(sz, 3),
            full(3, ch), full(1, ch), full(ch, ch), full(1, ch),
            full(ch, ch), full(1, ch), full(ch, g), full(1, g),
            full(g, ch), full(ch, ch), full(1, ch),
        ],
        out_specs=row(TN, ch),
        out_shape=jax.ShapeDtypeStruct((n, ch), jnp.float32),
    )(q, feat, coord, nloc, ktab, vtab, coord,
      p["pe1"]["w"], p["pe1"]["b"].reshape(1, -1),
      p["pe2"]["w"], p["pe2"]["b"].reshape(1, -1),
      p["we1"]["w"], p["we1"]["b"].reshape(1, -1),
      p["we2"]["w"], p["we2"]["b"].reshape(1, -1),
      gmat, p["proj"]["w"], p["proj"]["b"].reshape(1, -1))


def _cluskey_kern(cnt_ref, c_ref, caT_ref, key_ref, rep_ref, *, SZ, RT, GRID):
    b = pl.program_id(0)
    t = pl.program_id(1)
    cnt = cnt_ref[b]
    rows = c_ref[...]
    cx = caT_ref[0, 0:1, :]
    cy = caT_ref[0, 1:2, :]
    cz = caT_ref[0, 2:3, :]
    lane = jax.lax.broadcasted_iota(jnp.int32, (1, SZ), 1)
    lv = lane < cnt
    bmx = jnp.min(jnp.where(lv, cx, jnp.inf), axis=1, keepdims=True)
    bmy = jnp.min(jnp.where(lv, cy, jnp.inf), axis=1, keepdims=True)
    bmz = jnp.min(jnp.where(lv, cz, jnp.inf), axis=1, keepdims=True)
    kw = (jnp.floor((cx - bmx) / GRID).astype(jnp.int32) * 1000000
          + jnp.floor((cy - bmy) / GRID).astype(jnp.int32) * 1000
          + jnp.floor((cz - bmz) / GRID).astype(jnp.int32))
    kr = (jnp.floor((rows[:, 0:1] - bmx) / GRID).astype(jnp.int32) * 1000000
          + jnp.floor((rows[:, 1:2] - bmy) / GRID).astype(jnp.int32) * 1000
          + jnp.floor((rows[:, 2:3] - bmz) / GRID).astype(jnp.int32))
    rloc = t * RT + jax.lax.broadcasted_iota(jnp.int32, (RT, 1), 0)
    dup = lv & (lane < rloc) & (kw == kr)
    rep = jnp.logical_not(jnp.any(dup, axis=1, keepdims=True))
    rep = rep & (rloc < cnt)
    key_ref[...] = kr
    rep_ref[...] = rep.astype(jnp.int32)


def _clusrank_kern(cnt_ref, key_ref, keyT_ref, repT_ref, rank_ref, *, SZ):
    b = pl.program_id(0)
    cnt = cnt_ref[b]
    kr = key_ref[...]
    kw = keyT_ref[0, 0:1, :]
    rw = repT_ref[0, 0:1, :]
    lane = jax.lax.broadcasted_iota(jnp.int32, (1, SZ), 1)
    cmp = (lane < cnt) & (rw > 0) & (kw < kr)
    rank_ref[...] = jnp.sum(cmp.astype(jnp.int32), axis=1, keepdims=True)


def _cluster_aligned(cur_c, cnt_in, sz_in, sz_out, grid):
    mm = cur_c.shape[0]
    rt = min(128, sz_in)
    tt = sz_in // rt
    caT = cur_c.reshape(_B, sz_in, 3).transpose(0, 2, 1)
    key, rep = pl.pallas_call(
        functools.partial(_cluskey_kern, SZ=sz_in, RT=rt, GRID=grid),
        grid=(_B, tt),
        in_specs=[
            pl.BlockSpec(memory_space=pltpu.MemorySpace.SMEM),
            pl.BlockSpec((rt, 3), lambda b, t: (b * tt + t, 0)),
            pl.BlockSpec((1, 3, sz_in), lambda b, t: (b, 0, 0)),
        ],
        out_specs=[pl.BlockSpec((rt, 1), lambda b, t: (b * tt + t, 0)),
                   pl.BlockSpec((rt, 1), lambda b, t: (b * tt + t, 0))],
        out_shape=[jax.ShapeDtypeStruct((mm, 1), jnp.int32),
                   jax.ShapeDtypeStruct((mm, 1), jnp.int32)],
    )(cnt_in, cur_c, caT)
    keyT = key.reshape(_B, 1, sz_in)
    repT = rep.reshape(_B, 1, sz_in)
    rank = pl.pallas_call(
        functools.partial(_clusrank_kern, SZ=sz_in),
        grid=(_B, tt),
        in_specs=[
            pl.BlockSpec(memory_space=pltpu.MemorySpace.SMEM),
            pl.BlockSpec((rt, 1), lambda b, t: (b * tt + t, 0)),
            pl.BlockSpec((1, 1, sz_in), lambda b, t: (b, 0, 0)),
            pl.BlockSpec((1, 1, sz_in), lambda b, t: (b, 0, 0)),
        ],
        out_specs=pl.BlockSpec((rt, 1), lambda b, t: (b * tt + t, 0)),
        out_shape=jax.ShapeDtypeStruct((mm, 1), jnp.int32),
    )(cnt_in, key, keyT, repT)
    idx = jnp.arange(mm, dtype=jnp.int32)
    bid = idx // sz_in
    validv = (idx % sz_in) < cnt_in[bid]
    clus = jnp.where(validv, bid * sz_out + rank[:, 0], _B * sz_out)
    seg_bcnt = jnp.sum(rep.reshape(_B, sz_in), axis=1, dtype=jnp.int32)
    rk = rank[:, 0].reshape(_B, sz_in)
    lv = jnp.arange(sz_in)[None, :] < cnt_in[:, None]
    oh = ((rk[:, None, :] == jnp.arange(sz_out)[None, :, None])
          & lv[:, None, :]).astype(jnp.float32)
    sums = jnp.einsum('bsq,bqc->bsc', oh, cur_c.reshape(_B, sz_in, 3),
                      precision=jax.lax.Precision.HIGHEST,
                      preferred_element_type=jnp.float32)
    szs = jnp.sum(oh, axis=2)
    seg_valid = jnp.arange(sz_out)[None, :] < seg_bcnt[:, None]
    cnt = jnp.where(seg_valid, szs, 1.0)
    pooled = (sums / cnt[..., None]).reshape(_B * sz_out, 3)
    return (clus, seg_bcnt, pooled, cnt.reshape(-1),
            seg_valid.reshape(-1))


def _structures(coord, offset):
    n = coord.shape[0]
    bid0 = jnp.searchsorted(offset, jnp.arange(n), side="right").astype(jnp.int32)
    valid0 = jnp.ones(n, dtype=bool)
    starts0 = jnp.concatenate([jnp.zeros(1, jnp.int32), offset[:-1]])
    bcnt0 = offset - starts0
    nbr0 = _knn_pallas(coord, bcnt0, bid0, valid0, bcnt0, _PE_K, _SZ[0])
    stages = []
    cur_c, cur_v, cnt_in, sz_in = coord, valid0, bcnt0, _SZ[0]
    for i in range(4):
        sz = _SZ[i]
        m = _B * sz
        clus, seg_bcnt, pooled_c, cnt, seg_valid = _cluster_aligned(
            cur_c, cnt_in, sz_in, sz, _GRIDS[i])
        seg_bid = jnp.where(seg_valid,
                            jnp.arange(m, dtype=jnp.int32) // sz, 0)
        nbr = _knn_pallas(pooled_c, seg_bcnt, seg_bid, seg_valid, seg_bcnt,
                          _ENC_K[i], sz)
        stages.append((clus, m, cnt, nbr, seg_valid, pooled_c))
        cur_c, cur_v, cnt_in, sz_in = pooled_c, seg_valid, seg_bcnt, sz
    bid = jnp.where(cur_v, jnp.arange(_B * _SZ[3], dtype=jnp.int32) // _SZ[3], 0)
    bcnt = cnt_in.astype(jnp.float32)
    return nbr0, stages, bid, bcnt


def kernel(coord, feat, offset, params):
    nbr0, stages, bid, bcnt = _structures(coord, offset)
    x = coord
    f = _plinear(feat, params["embed"]["w"], params["embed"]["b"])
    f = _gva(params["pe_block"], f, x, nbr0, _PE_G, _SZ[0])
    for i in range(4):
        f = _plinear(f, params["down"][i]["w"], params["down"][i]["b"],
                     relu=True)
        clus, nseg, cnt, nbr, seg_valid, px = stages[i]
        pf = jax.ops.segment_max(f, clus, num_segments=nseg)
        pf = jnp.where(seg_valid[:, None], pf, 0.0)
        f = _gva(params["blocks"][i], pf, px, nbr, _ENC_G[i], _SZ[i])
        x = px
    f = jnp.where(stages[-1][4][:, None], f, 0.0)
    pooled = jax.ops.segment_sum(f, bid, num_segments=bcnt.shape[0]) / bcnt[:, None]
    return _phead(pooled, params["head1"], params["head2"], params["head3"])


# speedup vs baseline: 1.0900x; 1.0154x over previous
import functools

import jax
import jax.numpy as jnp
from jax.experimental import pallas as pl
from jax.experimental.pallas import tpu as pltpu

_B = 8
_PE_G = 6
_PE_K = 8
_ENC_CH = (96, 192, 384, 512)
_ENC_G = (12, 24, 48, 64)
_ENC_K = (16, 16, 16, 16)
_GRIDS = (0.06, 0.15, 0.375, 0.9375)
_SZ = (1024, 384, 32, 8)


def _linear_kern(x_ref, w_ref, b_ref, o_ref, *, relu):
    y = jnp.dot(x_ref[...], w_ref[...], preferred_element_type=jnp.float32)
    y = y + b_ref[...]
    if relu:
        y = jnp.maximum(y, 0.0)
    o_ref[...] = y


def _plinear(x, w, b, relu=False, tile=512):
    n, din = x.shape
    dout = w.shape[1]
    tile = min(tile, n)
    while n % tile:
        tile //= 2
    return pl.pallas_call(
        functools.partial(_linear_kern, relu=relu),
        grid=(n // tile,),
        in_specs=[
            pl.BlockSpec((tile, din), lambda i: (i, 0)),
            pl.BlockSpec((din, dout), lambda i: (0, 0)),
            pl.BlockSpec((1, dout), lambda i: (0, 0)),
        ],
        out_specs=pl.BlockSpec((tile, dout), lambda i: (i, 0)),
        out_shape=jax.ShapeDtypeStruct((n, dout), jnp.float32),
    )(x, w, b.reshape(1, -1))


def _head_kern(x_ref, w1_ref, b1_ref, w2_ref, b2_ref, w3_ref, b3_ref, o_ref):
    h = jnp.dot(x_ref[...], w1_ref[...], preferred_element_type=jnp.float32)
    h = jnp.maximum(h + b1_ref[...], 0.0)
    h = jnp.dot(h, w2_ref[...], preferred_element_type=jnp.float32)
    h = jnp.maximum(h + b2_ref[...], 0.0)
    h = jnp.dot(h, w3_ref[...], preferred_element_type=jnp.float32)
    o_ref[...] = h + b3_ref[...]


def _phead(x, p1, p2, p3):
    n = x.shape[0]
    dout = p3["w"].shape[1]
    return pl.pallas_call(
        _head_kern,
        out_shape=jax.ShapeDtypeStruct((n, dout), jnp.float32),
    )(x, p1["w"], p1["b"].reshape(1, -1), p2["w"], p2["b"].reshape(1, -1),
      p3["w"], p3["b"].reshape(1, -1))


def _knn_kern(cnt_ref, c_ref, caT_ref, o_ref, *, K, SZ):
    b = pl.program_id(0)
    cnt = cnt_ref[b]
    rows = c_ref[...]
    RT = rows.shape[0]
    cx = caT_ref[0, 0:1, :]
    cy = caT_ref[0, 1:2, :]
    cz = caT_ref[0, 2:3, :]
    dx = rows[:, 0:1] - cx
    dy = rows[:, 1:2] - cy
    dz = rows[:, 2:3] - cz
    d = (dx * dx + dy * dy) + dz * dz
    jl = jax.lax.broadcasted_iota(jnp.int32, (RT, SZ), 1)
    d = jnp.where(jl < cnt, d, jnp.inf)
    cols = []
    for _ in range(K):
        m = jnp.min(d, axis=1, keepdims=True)
        ismin = d == m
        fidx = jnp.min(jnp.where(ismin, jl, SZ), axis=1, keepdims=True)
        cols.append(fidx)
        d = jnp.where(jl == fidx, jnp.inf, d)
    out = jnp.concatenate(cols, axis=1).astype(jnp.int32) + b * SZ
    o_ref[...] = out


def _knn_pallas(c, cnts, bidv, validv, bcount, k, sz):
    m = c.shape[0]
    rt = min(256, sz)
    while sz % rt:
        rt //= 2
    tt = sz // rt
    caT = c.reshape(_B, sz, 3).transpose(0, 2, 1)
    nb = pl.pallas_call(
        functools.partial(_knn_kern, K=k, SZ=sz),
        grid=(_B, tt),
        in_specs=[
            pl.BlockSpec(memory_space=pltpu.MemorySpace.SMEM),
            pl.BlockSpec((rt, 3), lambda b, t: (b * tt + t, 0)),
            pl.BlockSpec((1, 3, sz), lambda b, t: (b, 0, 0)),
        ],
        out_specs=pl.BlockSpec((rt, k), lambda b, t: (b * tt + t, 0)),
        out_shape=jax.ShapeDtypeStruct((m, k), jnp.int32),
    )(cnts, c, caT)
    vcnt = bcount[jnp.where(validv, bidv, 0)]
    col = jnp.arange(k)[None, :]
    last = jnp.take_along_axis(nb, jnp.clip(vcnt - 1, 0, k - 1)[:, None], axis=1)
    return jnp.where(col < vcnt[:, None], nb, last)


def _gva_kern(q_ref, feat_ref, co_ref, nbr_ref, ktab_ref, vtab_ref, cw_ref,
              pe1_ref, pe1b_ref, pe2_ref, pe2b_ref,
              we1_ref, we1b_ref, we2_ref, we2b_ref,
              gmat_ref, proj_ref, projb_ref, o_ref, *, K, G, SZ):
    TN, ch = q_ref.shape
    TNK = TN * K
    nl = nbr_ref[...]
    oh = (jax.lax.broadcasted_iota(jnp.int32, (TNK, SZ), 1) == nl)
    oh = oh.astype(jnp.float32)
    kn = jnp.dot(oh, ktab_ref[...], preferred_element_type=jnp.float32)
    vn = jnp.dot(oh, vtab_ref[...], preferred_element_type=jnp.float32)
    cn = jnp.dot(oh, cw_ref[...], preferred_element_type=jnp.float32)
    pos = (cn.reshape(TN, K, 3) - co_ref[...].reshape(TN, 1, 3)).reshape(TNK, 3)
    h = jnp.dot(pos, pe1_ref[...], preferred_element_type=jnp.float32)
    h = jnp.maximum(h + pe1b_ref[...], 0.0)
    pe = jnp.dot(h, pe2_ref[...], preferred_element_type=jnp.float32)
    pe = pe + pe2b_ref[...]
    rel = (q_ref[...].reshape(TN, 1, ch)
           - kn.reshape(TN, K, ch)
           + pe.reshape(TN, K, ch)).reshape(TNK, ch)
    t = jnp.dot(rel, we1_ref[...], preferred_element_type=jnp.float32)
    t = jnp.maximum(t + we1b_ref[...], 0.0)
    w = jnp.dot(t, we2_ref[...], preferred_element_type=jnp.float32)
    w = (w + we2b_ref[...]).reshape(TN, K, G)
    w = w - jnp.max(w, axis=1, keepdims=True)
    e = jnp.exp(w)
    w = e / jnp.sum(e, axis=1, keepdims=True)
    wfull = jnp.dot(w.reshape(TNK, G), gmat_ref[...],
                    preferred_element_type=jnp.float32)
    val = vn + pe
    out = jnp.sum((wfull * val).reshape(TN, K, ch), axis=1)
    y = jnp.dot(out, proj_ref[...], preferred_element_type=jnp.float32)
    y = y + projb_ref[...]
    o_ref[...] = feat_ref[...] + jnp.maximum(y, 0.0)


def _gva(p, feat, coord, nbr, g, sz):
    n, ch = feat.shape
    k = nbr.shape[1]
    wqkv = jnp.concatenate([p["q"]["w"], p["k"]["w"], p["v"]["w"]], axis=1)
    bqkv = jnp.concatenate([p["q"]["b"], p["k"]["b"], p["v"]["b"]])
    qkv = _plinear(feat, wqkv, bqkv)
    q = qkv[:, :ch]
    ktab = qkv[:, ch:2 * ch]
    vtab = qkv[:, 2 * ch:]
    nloc = (nbr - (jnp.arange(n, dtype=jnp.int32)[:, None] // sz) * sz
            ).reshape(n * k, 1)
    gmat = (jnp.arange(ch)[None, :] // (ch // g)
            == jnp.arange(g)[:, None]).astype(jnp.float32)
    TN = min(256, sz)
    while sz % TN:
        TN //= 2
    tpb = sz // TN
    row = lambda r, c: pl.BlockSpec((r, c), lambda i: (i, 0))
    win = lambda r, c: pl.BlockSpec((r, c), lambda i: (i // tpb, 0))
    full = lambda r, c: pl.BlockSpec((r, c), lambda i: (0, 0))
    return pl.pallas_call(
        functools.partial(_gva_kern, K=k, G=g, SZ=sz),
        grid=(n // TN,),
        in_specs=[
            row(TN, ch), row(TN, ch), row(TN, 3), row(TN * k, 1),
            win(sz, ch), win(sz, ch), win(sz, 3),
            full(3, ch), full(1, ch), full(ch, ch), full(1, ch),
            full(ch, ch), full(1, ch), full(ch, g), full(1, g),
            full(g, ch), full(ch, ch), full(1, ch),
        ],
        out_specs=row(TN, ch),
        out_shape=jax.ShapeDtypeStruct((n, ch), jnp.float32),
    )(q, feat, coord, nloc, ktab, vtab, coord,
      p["pe1"]["w"], p["pe1"]["b"].reshape(1, -1),
      p["pe2"]["w"], p["pe2"]["b"].reshape(1, -1),
      p["we1"]["w"], p["we1"]["b"].reshape(1, -1),
      p["we2"]["w"], p["we2"]["b"].reshape(1, -1),
      gmat, p["proj"]["w"], p["proj"]["b"].reshape(1, -1))


def _cluskey_kern(cnt_ref, c_ref, caT_ref, key_ref, rep_ref, *, SZ, RT, GRID):
    b = pl.program_id(0)
    t = pl.program_id(1)
    cnt = cnt_ref[b]
    rows = c_ref[...]
    cx = caT_ref[0, 0:1, :]
    cy = caT_ref[0, 1:2, :]
    cz = caT_ref[0, 2:3, :]
    lane = jax.lax.broadcasted_iota(jnp.int32, (1, SZ), 1)
    lv = lane < cnt
    bmx = jnp.min(jnp.where(lv, cx, jnp.inf), axis=1, keepdims=True)
    bmy = jnp.min(jnp.where(lv, cy, jnp.inf), axis=1, keepdims=True)
    bmz = jnp.min(jnp.where(lv, cz, jnp.inf), axis=1, keepdims=True)
    kw = (jnp.floor((cx - bmx) / GRID).astype(jnp.int32) * 1000000
          + jnp.floor((cy - bmy) / GRID).astype(jnp.int32) * 1000
          + jnp.floor((cz - bmz) / GRID).astype(jnp.int32))
    kr = (jnp.floor((rows[:, 0:1] - bmx) / GRID).astype(jnp.int32) * 1000000
          + jnp.floor((rows[:, 1:2] - bmy) / GRID).astype(jnp.int32) * 1000
          + jnp.floor((rows[:, 2:3] - bmz) / GRID).astype(jnp.int32))
    rloc = t * RT + jax.lax.broadcasted_iota(jnp.int32, (RT, 1), 0)
    dup = lv & (lane < rloc) & (kw == kr)
    rep = jnp.logical_not(jnp.any(dup, axis=1, keepdims=True))
    rep = rep & (rloc < cnt)
    key_ref[...] = kr
    rep_ref[...] = rep.astype(jnp.int32)


def _clusrank_kern(cnt_ref, key_ref, keyT_ref, repT_ref, rank_ref, *, SZ):
    b = pl.program_id(0)
    cnt = cnt_ref[b]
    kr = key_ref[...]
    kw = keyT_ref[0, 0:1, :]
    rw = repT_ref[0, 0:1, :]
    lane = jax.lax.broadcasted_iota(jnp.int32, (1, SZ), 1)
    cmp = (lane < cnt) & (rw > 0) & (kw < kr)
    rank_ref[...] = jnp.sum(cmp.astype(jnp.int32), axis=1, keepdims=True)


def _cluster_aligned(cur_c, cnt_in, sz_in, sz_out, grid):
    mm = cur_c.shape[0]
    rt = min(128, sz_in)
    tt = sz_in // rt
    caT = cur_c.reshape(_B, sz_in, 3).transpose(0, 2, 1)
    key, rep = pl.pallas_call(
        functools.partial(_cluskey_kern, SZ=sz_in, RT=rt, GRID=grid),
        grid=(_B, tt),
        in_specs=[
            pl.BlockSpec(memory_space=pltpu.MemorySpace.SMEM),
            pl.BlockSpec((rt, 3), lambda b, t: (b * tt + t, 0)),
            pl.BlockSpec((1, 3, sz_in), lambda b, t: (b, 0, 0)),
        ],
        out_specs=[pl.BlockSpec((rt, 1), lambda b, t: (b * tt + t, 0)),
                   pl.BlockSpec((rt, 1), lambda b, t: (b * tt + t, 0))],
        out_shape=[jax.ShapeDtypeStruct((mm, 1), jnp.int32),
                   jax.ShapeDtypeStruct((mm, 1), jnp.int32)],
    )(cnt_in, cur_c, caT)
    keyT = key.reshape(_B, 1, sz_in)
    repT = rep.reshape(_B, 1, sz_in)
    rank = pl.pallas_call(
        functools.partial(_clusrank_kern, SZ=sz_in),
        grid=(_B, tt),
        in_specs=[
            pl.BlockSpec(memory_space=pltpu.MemorySpace.SMEM),
            pl.BlockSpec((rt, 1), lambda b, t: (b * tt + t, 0)),
            pl.BlockSpec((1, 1, sz_in), lambda b, t: (b, 0, 0)),
            pl.BlockSpec((1, 1, sz_in), lambda b, t: (b, 0, 0)),
        ],
        out_specs=pl.BlockSpec((rt, 1), lambda b, t: (b * tt + t, 0)),
        out_shape=jax.ShapeDtypeStruct((mm, 1), jnp.int32),
    )(cnt_in, key, keyT, repT)
    idx = jnp.arange(mm, dtype=jnp.int32)
    bid = idx // sz_in
    validv = (idx % sz_in) < cnt_in[bid]
    clus = jnp.where(validv, bid * sz_out + rank[:, 0], _B * sz_out)
    seg_bcnt = jnp.sum(rep.reshape(_B, sz_in), axis=1, dtype=jnp.int32)
    rk = rank[:, 0].reshape(_B, sz_in)
    lv = jnp.arange(sz_in)[None, :] < cnt_in[:, None]
    oh = ((rk[:, None, :] == jnp.arange(sz_out)[None, :, None])
          & lv[:, None, :]).astype(jnp.float32)
    sums = jnp.einsum('bsq,bqc->bsc', oh, cur_c.reshape(_B, sz_in, 3),
                      precision=jax.lax.Precision.HIGHEST,
                      preferred_element_type=jnp.float32)
    szs = jnp.sum(oh, axis=2)
    seg_valid = jnp.arange(sz_out)[None, :] < seg_bcnt[:, None]
    cnt = jnp.where(seg_valid, szs, 1.0)
    pooled = (sums / cnt[..., None]).reshape(_B * sz_out, 3)
    return (clus, seg_bcnt, pooled, cnt.reshape(-1),
            seg_valid.reshape(-1))


def _structures(coord, offset):
    n = coord.shape[0]
    bid0 = jnp.searchsorted(offset, jnp.arange(n), side="right").astype(jnp.int32)
    valid0 = jnp.ones(n, dtype=bool)
    starts0 = jnp.concatenate([jnp.zeros(1, jnp.int32), offset[:-1]])
    bcnt0 = offset - starts0
    nbr0 = _knn_pallas(coord, bcnt0, bid0, valid0, bcnt0, _PE_K, _SZ[0])
    stages = []
    cur_c, cur_v, cnt_in, sz_in = coord, valid0, bcnt0, _SZ[0]
    for i in range(4):
        sz = _SZ[i]
        m = _B * sz
        clus, seg_bcnt, pooled_c, cnt, seg_valid = _cluster_aligned(
            cur_c, cnt_in, sz_in, sz, _GRIDS[i])
        seg_bid = jnp.where(seg_valid,
                            jnp.arange(m, dtype=jnp.int32) // sz, 0)
        nbr = _knn_pallas(pooled_c, seg_bcnt, seg_bid, seg_valid, seg_bcnt,
                          _ENC_K[i], sz)
        stages.append((clus, m, cnt, nbr, seg_valid, pooled_c))
        cur_c, cur_v, cnt_in, sz_in = pooled_c, seg_valid, seg_bcnt, sz
    bid = jnp.where(cur_v, jnp.arange(_B * _SZ[3], dtype=jnp.int32) // _SZ[3], 0)
    bcnt = cnt_in.astype(jnp.float32)
    return nbr0, stages, bid, bcnt


def kernel(coord, feat, offset, params):
    nbr0, stages, bid, bcnt = _structures(coord, offset)
    x = coord
    f = _plinear(feat, params["embed"]["w"], params["embed"]["b"])
    f = _gva(params["pe_block"], f, x, nbr0, _PE_G, _SZ[0])
    for i in range(4):
        f = _plinear(f, params["down"][i]["w"], params["down"][i]["b"],
                     relu=True)
        clus, nseg, cnt, nbr, seg_valid, px = stages[i]
        pf = jax.ops.segment_max(f, clus, num_segments=nseg)
        pf = jnp.where(seg_valid[:, None], pf, 0.0)
        f = _gva(params["blocks"][i], pf, px, nbr, _ENC_G[i], _SZ[i])
        x = px
    f = jnp.where(stages[-1][4][:, None], f, 0.0)
    pooled = jax.ops.segment_sum(f, bid, num_segments=bcnt.shape[0]) / bcnt[:, None]
    return _phead(pooled, params["head1"], params["head2"], params["head3"])
